# baseline (speedup 1.0000x reference)
import functools
import math

import jax
import jax.numpy as jnp
from jax import lax
from jax.experimental import pallas as pl
from jax.experimental.pallas import tpu as pltpu
from jax.experimental.pallas import tpu_sc as plsc

EMB = 64
SCALE = math.sqrt(EMB)

NC = 2
NS = 16
NW = NC * NS
LANES = 16

CH = 128
NBUF = 4


def _make_gather(R, T):
    B = R * T
    assert B % (NW * CH) == 0
    b_per_w = B // NW
    nchunk = b_per_w // CH
    mesh = plsc.VectorSubcoreMesh(
        core_axis_name="c", subcore_axis_name="s", num_cores=NC, num_subcores=NS
    )

    @functools.partial(
        pl.kernel,
        out_type=jax.ShapeDtypeStruct((R * T, 2 * EMB), jnp.float32),
        mesh=mesh,
        compiler_params=pltpu.CompilerParams(
            use_tc_tiling_on_sc=False, needs_layout_passes=False),
        scratch_types=[
            pltpu.VMEM((nchunk // 2, CH), jnp.int32),
            pltpu.VMEM((nchunk // 2, CH), jnp.int32),
            pltpu.VMEM((NBUF, CH, 2 * EMB), jnp.float32),
            pltpu.VMEM((NBUF, CH, EMB), jnp.float32),
            pltpu.SemaphoreType.DMA((NBUF,)),
            pltpu.SemaphoreType.DMA((NBUF,)),
        ],
    )
    def gather_kernel(pair_hbm, off_hbm, table_hbm, out_hbm, idx_v, off_v,
                      gbuf, sbuf, gsem, ssem):
        wid = lax.axis_index("s") * NC + lax.axis_index("c")
        base = wid * b_per_w
        half = nchunk // 2

        def gather_copy(j, b):
            return pltpu.make_async_copy(
                table_hbm.at[idx_v.at[j]], gbuf.at[b], gsem.at[b])

        def store_copy(h, j, b):
            return pltpu.make_async_copy(
                sbuf.at[b],
                out_hbm.at[pl.ds(base + (h * half + j) * CH, CH),
                           pl.ds(0, EMB)],
                ssem.at[b])

        for h in range(2):
            pltpu.sync_copy(pair_hbm.at[wid, pl.ds(h * half, half)], idx_v)
            pltpu.sync_copy(off_hbm.at[wid, pl.ds(h * half, half)], off_v)

            for b in range(NBUF):
                gather_copy(b, b).start()

            @pl.loop(0, half, step=NBUF)
            def _group(g):
                for b in range(NBUF):
                    j = g + b
                    gather_copy(j, b).wait()

                    @pl.when(j >= NBUF)
                    def _drain():
                        store_copy(h, j - NBUF, b).wait()

                    src = gbuf.at[b]
                    dst = sbuf.at[b]

                    @pl.loop(0, CH // LANES)
                    def _extract(grp):
                        rows = lax.iota(jnp.int32, LANES) + grp * LANES
                        offs = off_v[j, pl.ds(grp * LANES, LANES)]

                        @pl.loop(0, EMB, unroll=4)
                        def _col(c):
                            v = plsc.load_gather(src, [rows, offs + c])
                            plsc.store_scatter(dst, [rows, rows * 0 + c],
                                               v * SCALE)

                    nj = j + NBUF

                    @pl.when(nj < half)
                    def _prefetch():
                        gather_copy(nj, b).start()

                    store_copy(h, j, b).start()

            for b in range(NBUF):
                store_copy(h, half - NBUF + b, b).wait()

    return gather_kernel


def kernel(tokens, table):
    R, T = tokens.shape
    B = R * T
    tok = tokens.astype(jnp.int32).reshape(NW, B // NW // CH, CH)
    pair = tok >> 1
    off = (tok & 1) * EMB
    table2 = table.reshape(table.shape[0] // 2, 2 * EMB)
    padded = _make_gather(R, T)(pair, off, table2)
    return padded.reshape(R, T, 2 * EMB)[:, :, :EMB]


# speedup vs baseline: 2.4880x; 2.4880x over previous
import functools
import math

import jax
import jax.numpy as jnp
from jax import lax
from jax.experimental import pallas as pl
from jax.experimental.pallas import tpu as pltpu
from jax.experimental.pallas import tpu_sc as plsc

EMB = 64
SCALE = math.sqrt(EMB)

NC = 2
NS = 16
NW = NC * NS
LANES = 16

CH = 128
NBUF = 4


def _make_gather(R, T):
    B = R * T
    assert B % (NW * CH) == 0
    b_per_w = B // NW
    nchunk = b_per_w // CH
    mesh = plsc.VectorSubcoreMesh(
        core_axis_name="c", subcore_axis_name="s", num_cores=NC, num_subcores=NS
    )

    @functools.partial(
        pl.kernel,
        out_type=jax.ShapeDtypeStruct((R * T, 2 * EMB), jnp.float32),
        mesh=mesh,
        compiler_params=pltpu.CompilerParams(
            use_tc_tiling_on_sc=False, needs_layout_passes=False),
        scratch_types=[
            pltpu.VMEM((nchunk // 2, CH), jnp.int32),
            pltpu.VMEM((nchunk // 2, CH), jnp.int32),
            pltpu.VMEM((NBUF, CH, 2 * EMB), jnp.float32),
            pltpu.VMEM((NBUF, CH, EMB), jnp.float32),
            pltpu.SemaphoreType.DMA((NBUF,)),
            pltpu.SemaphoreType.DMA((NBUF,)),
        ],
    )
    def gather_kernel(pair_hbm, off_hbm, table_hbm, out_hbm, idx_v, off_v,
                      gbuf, sbuf, gsem, ssem):
        wid = lax.axis_index("s") * NC + lax.axis_index("c")
        base = wid * b_per_w
        half = nchunk // 2

        def gather_copy(j, b):
            return pltpu.make_async_copy(
                table_hbm.at[idx_v.at[j]], gbuf.at[b], gsem.at[b])

        def store_copy(h, j, b):
            return pltpu.make_async_copy(
                sbuf.at[b],
                out_hbm.at[pl.ds(base + (h * half + j) * CH, CH),
                           pl.ds(0, EMB)],
                ssem.at[b])

        for h in range(2):
            pltpu.sync_copy(pair_hbm.at[wid, pl.ds(h * half, half)], idx_v)
            pltpu.sync_copy(off_hbm.at[wid, pl.ds(h * half, half)], off_v)

            for b in range(NBUF):
                gather_copy(b, b).start()

            @pl.loop(0, half, step=NBUF)
            def _group(g):
                for b in range(NBUF):
                    j = g + b
                    gather_copy(j, b).wait()

                    @pl.when(j >= NBUF)
                    def _drain():
                        store_copy(h, j - NBUF, b).wait()

                    src = gbuf.at[b]
                    dst = sbuf.at[b]

                    @plsc.parallel_loop(0, CH // LANES)
                    def _extract(grp):
                        offs = off_v[j, pl.ds(grp * LANES, LANES)]
                        for l in range(LANES):
                            k = grp * LANES + l
                            off_l = offs[l]
                            for c in range(EMB // LANES):
                                dst[k, pl.ds(c * LANES, LANES)] = (
                                    src[k, pl.ds(off_l + c * LANES, LANES)]
                                    * SCALE)

                    nj = j + NBUF

                    @pl.when(nj < half)
                    def _prefetch():
                        gather_copy(nj, b).start()

                    store_copy(h, j, b).start()

            for b in range(NBUF):
                store_copy(h, half - NBUF + b, b).wait()

    return gather_kernel


def kernel(tokens, table):
    R, T = tokens.shape
    B = R * T
    tok = tokens.astype(jnp.int32).reshape(NW, B // NW // CH, CH)
    pair = tok >> 1
    off = (tok & 1) * EMB
    table2 = table.reshape(table.shape[0] // 2, 2 * EMB)
    padded = _make_gather(R, T)(pair, off, table2)
    return padded.reshape(R, T, 2 * EMB)[:, :, :EMB]


# speedup vs baseline: 2.9961x; 1.2042x over previous
import functools
import math

import jax
import jax.numpy as jnp
from jax import lax
from jax.experimental import pallas as pl
from jax.experimental.pallas import tpu as pltpu
from jax.experimental.pallas import tpu_sc as plsc

EMB = 64
SCALE = math.sqrt(EMB)

NC = 2
NS = 16
NW = NC * NS
LANES = 16

CH = 128
NBUF = 4


def _make_gather(B):
    assert B % (NW * CH) == 0
    b_per_w = B // NW
    nchunk = b_per_w // CH
    mesh = plsc.VectorSubcoreMesh(
        core_axis_name="c", subcore_axis_name="s", num_cores=NC, num_subcores=NS
    )

    @functools.partial(
        pl.kernel,
        out_type=jax.ShapeDtypeStruct((B, 2 * EMB), jnp.float32),
        mesh=mesh,
        compiler_params=pltpu.CompilerParams(
            use_tc_tiling_on_sc=False, needs_layout_passes=False),
        scratch_types=[
            pltpu.VMEM((nchunk, CH), jnp.int32),
            pltpu.VMEM((NBUF, CH, EMB), jnp.float32),
            pltpu.VMEM((NBUF, CH, EMB), jnp.float32),
            pltpu.SemaphoreType.DMA((NBUF,)),
            pltpu.SemaphoreType.DMA((NBUF,)),
        ],
    )
    def gather_kernel(tok_hbm, table_hbm, out_hbm, idx_v, gbuf, sbuf,
                      gsem, ssem):
        wid = lax.axis_index("s") * NC + lax.axis_index("c")
        base = wid * b_per_w
        pltpu.sync_copy(tok_hbm.at[wid], idx_v)

        def gather_copy(j, b):
            return pltpu.make_async_copy(
                table_hbm.at[idx_v.at[j]], gbuf.at[b], gsem.at[b])

        def store_copy(j, b):
            return pltpu.make_async_copy(
                sbuf.at[b],
                out_hbm.at[pl.ds(base + j * CH, CH), pl.ds(0, EMB)],
                ssem.at[b])

        for b in range(NBUF):
            gather_copy(b, b).start()

        @pl.loop(0, nchunk, step=NBUF)
        def _group(g):
            for b in range(NBUF):
                j = g + b
                gather_copy(j, b).wait()

                @pl.when(j >= NBUF)
                def _drain():
                    store_copy(j - NBUF, b).wait()

                src = gbuf.at[b]
                dst = sbuf.at[b]

                @plsc.parallel_loop(0, CH, unroll=4)
                def _scale(r):
                    for c in range(EMB // LANES):
                        sl = pl.ds(c * LANES, LANES)
                        dst[r, sl] = src[r, sl] * SCALE

                nj = j + NBUF

                @pl.when(nj < nchunk)
                def _prefetch():
                    gather_copy(nj, b).start()

                store_copy(j, b).start()

        for b in range(NBUF):
            store_copy(nchunk - NBUF + b, b).wait()

    return gather_kernel


def kernel(tokens, table):
    R, T = tokens.shape
    B = R * T
    tok = tokens.astype(jnp.int32).reshape(NW, B // NW // CH, CH)
    padded = _make_gather(B)(tok, table)
    return padded.reshape(R, T, 2 * EMB)[:, :, :EMB]
